# baseline (speedup 1.0000x reference)
import functools

import jax
import jax.numpy as jnp
from jax import lax
from jax.experimental import pallas as pl
from jax.experimental.pallas import tpu as pltpu
from jax.experimental.pallas import tpu_sc as plsc

VOCAB = 512
SEQ = 50
BATCH = 1024
A0 = 1.0 / 100.0
A1 = 39.0 / 100.0
A2 = 6.0 / 10.0

NC = 2
NS = 16
L = 16
NW = NC * NS
BPW = BATCH // NW
D = VOCAB

BB = 256
NB = BATCH // BB


def _sc_body(tokens_hbm, tri_hbm, out_hbm, prev_v, last_v, idx_v, tri_v, sem):
    wid = lax.axis_index("s") * NC + lax.axis_index("c")
    base = wid * BPW

    pltpu.sync_copy(tokens_hbm.at[SEQ - 2, pl.ds(base, BPW)], prev_v)
    pltpu.sync_copy(tokens_hbm.at[SEQ - 1, pl.ds(base, BPW)], last_v)

    for c in range(BPW // L):
        sl = pl.ds(c * L, L)
        idx_v[sl] = prev_v[sl] * VOCAB + last_v[sl]

    cp = pltpu.make_async_copy(tri_hbm.at[idx_v], tri_v, sem)
    cp.start()
    cp.wait()
    pltpu.sync_copy(tri_v, out_hbm.at[pl.ds(base, BPW)])


def _tc_partial_body(last_ref, bigram_ref, out_ref):
    toks = last_ref[0, 0, :]
    row_ids = jax.lax.broadcasted_iota(jnp.int32, (BB, VOCAB), 1)
    onehot = jnp.where(toks[:, None] == row_ids, jnp.float32(A1),
                       jnp.float32(0.0))
    out_ref[...] = jnp.dot(onehot, bigram_ref[...],
                           preferred_element_type=jnp.float32)


def _tc_blend_body(partial_ref, tri_ref, uni_ref, out_ref):
    out_ref[...] = (partial_ref[...] + A2 * tri_ref[...]
                    + A0 * uni_ref[0, :][None, :])


@jax.jit
def kernel(input_data, unigram_probs, bigram_probs, trigram_probs):
    tri2d = trigram_probs.reshape(VOCAB * VOCAB, VOCAB)
    last_3d = input_data[SEQ - 1].reshape(NB, 1, BB)

    mesh = plsc.VectorSubcoreMesh(
        core_axis_name="c", subcore_axis_name="s",
        num_cores=NC, num_subcores=NS,
    )
    sc_gather = pl.kernel(
        _sc_body,
        out_type=jax.ShapeDtypeStruct((BATCH, D), jnp.float32),
        mesh=mesh,
        scratch_types=[
            pltpu.VMEM((BPW,), jnp.int32),
            pltpu.VMEM((BPW,), jnp.int32),
            pltpu.VMEM((BPW,), jnp.int32),
            pltpu.VMEM((BPW, D), jnp.float32),
            pltpu.SemaphoreType.DMA,
        ],
    )
    tri_rows = sc_gather(input_data, tri2d)

    partial = pl.pallas_call(
        _tc_partial_body,
        grid=(NB,),
        in_specs=[
            pl.BlockSpec((1, 1, BB), lambda i: (i, 0, 0)),
            pl.BlockSpec((VOCAB, VOCAB), lambda i: (0, 0)),
        ],
        out_specs=pl.BlockSpec((BB, VOCAB), lambda i: (i, 0)),
        out_shape=jax.ShapeDtypeStruct((BATCH, VOCAB), jnp.float32),
    )(last_3d, bigram_probs)

    out = pl.pallas_call(
        _tc_blend_body,
        grid=(NB,),
        in_specs=[
            pl.BlockSpec((BB, VOCAB), lambda i: (i, 0)),
            pl.BlockSpec((BB, VOCAB), lambda i: (i, 0)),
            pl.BlockSpec((1, VOCAB), lambda i: (0, 0)),
        ],
        out_specs=pl.BlockSpec((BB, VOCAB), lambda i: (i, 0)),
        out_shape=jax.ShapeDtypeStruct((BATCH, VOCAB), jnp.float32),
    )(partial, tri_rows, unigram_probs.reshape(1, VOCAB))
    return out


# speedup vs baseline: 1.0471x; 1.0471x over previous
import jax
import jax.numpy as jnp
from jax import lax
from jax.experimental import pallas as pl
from jax.experimental.pallas import tpu as pltpu
from jax.experimental.pallas import tpu_sc as plsc

VOCAB = 512
SEQ = 50
BATCH = 1024
A0 = 1.0 / 100.0
A1 = 39.0 / 100.0
A2 = 6.0 / 10.0

NC = 2
NS = 16
L = 16
NW = NC * NS
BPW = BATCH // NW
D = VOCAB
NCHUNK = D // L
HALF = BPW // 2


def _body(tokens_hbm, uni_hbm, bi_hbm, tri_hbm, out_hbm,
          prev_v, last_v, idx_v, uni_v, bi_v, tri_v,
          bi_sem_a, bi_sem_b, tri_sem_a, tri_sem_b, out_sem):
    wid = lax.axis_index("s") * NC + lax.axis_index("c")
    base = wid * BPW

    pltpu.sync_copy(tokens_hbm.at[SEQ - 2, pl.ds(base, BPW)], prev_v)
    pltpu.sync_copy(tokens_hbm.at[SEQ - 1, pl.ds(base, BPW)], last_v)

    for c in range(BPW // L):
        sl = pl.ds(c * L, L)
        idx_v[sl] = prev_v[sl] * VOCAB + last_v[sl]

    copies = []
    for h, (bi_sem, tri_sem) in enumerate(((bi_sem_a, tri_sem_a),
                                           (bi_sem_b, tri_sem_b))):
        r0 = h * HALF
        rows = pl.ds(r0, HALF)
        cp_bi = pltpu.make_async_copy(
            bi_hbm.at[last_v.at[rows]], bi_v.at[rows], bi_sem)
        cp_tri = pltpu.make_async_copy(
            tri_hbm.at[idx_v.at[rows]], tri_v.at[rows], tri_sem)
        cp_bi.start()
        cp_tri.start()
        copies.append((cp_bi, cp_tri))

    pltpu.sync_copy(uni_hbm, uni_v)
    uni_c = [uni_v[pl.ds(c * L, L)] * A0 for c in range(NCHUNK)]

    def row(r, carry):
        for c in range(NCHUNK):
            sl = pl.ds(c * L, L)
            bi_v[r, sl] = uni_c[c] + A1 * bi_v[r, sl] + A2 * tri_v[r, sl]
        return carry

    out_copies = []
    for h, (cp_bi, cp_tri) in enumerate(copies):
        r0 = h * HALF
        cp_bi.wait()
        cp_tri.wait()
        lax.fori_loop(r0, r0 + HALF, row, 0)
        cp_out = pltpu.make_async_copy(
            bi_v.at[pl.ds(r0, HALF)],
            out_hbm.at[pl.ds(base + r0, HALF)],
            out_sem,
        )
        cp_out.start()
        out_copies.append(cp_out)
    for cp_out in out_copies:
        cp_out.wait()


@jax.jit
def kernel(input_data, unigram_probs, bigram_probs, trigram_probs):
    tri2d = trigram_probs.reshape(VOCAB * VOCAB, VOCAB)

    mesh = plsc.VectorSubcoreMesh(
        core_axis_name="c", subcore_axis_name="s",
        num_cores=NC, num_subcores=NS,
    )
    run = pl.kernel(
        _body,
        out_type=jax.ShapeDtypeStruct((BATCH, D), jnp.float32),
        mesh=mesh,
        scratch_types=[
            pltpu.VMEM((BPW,), jnp.int32),
            pltpu.VMEM((BPW,), jnp.int32),
            pltpu.VMEM((BPW,), jnp.int32),
            pltpu.VMEM((D,), jnp.float32),
            pltpu.VMEM((BPW, D), jnp.float32),
            pltpu.VMEM((BPW, D), jnp.float32),
            pltpu.SemaphoreType.DMA,
            pltpu.SemaphoreType.DMA,
            pltpu.SemaphoreType.DMA,
            pltpu.SemaphoreType.DMA,
            pltpu.SemaphoreType.DMA,
        ],
    )
    return run(input_data, unigram_probs, bigram_probs, tri2d)
